# Initial kernel scaffold; baseline (speedup 1.0000x reference)
import jax
import jax.numpy as jnp
from jax.experimental import pallas as pl


def kernel(h0, h1, h2, W_self_0, W_neigh_0, W_self_1, W_neigh_1):
    raise NotImplementedError("write your pallas kernel here")


# speedup vs baseline: 1.5845x; 1.5845x over previous
import functools

import jax
import jax.numpy as jnp
from jax.experimental import pallas as pl
from jax.experimental.pallas import tpu as pltpu

B = 2048
N0 = 10
N1 = 10
D = 256
R = 64


def _fused_sage_kernel(h0_ref, h1f_ref, h1w_ref, h2w_ref, w0_ref, w1_ref,
                       m_ref, out_ref):
    h2w = h2w_ref[...]
    m2 = h2w[:, 0:D]
    for k in range(1, N1):
        m2 = m2 + h2w[:, k * D:(k + 1) * D]
    m2 = m2 * (1.0 / N1)

    x1 = jnp.concatenate([h1f_ref[...], m2], axis=1)
    out1 = jnp.maximum(
        jnp.dot(x1, w0_ref[...], preferred_element_type=jnp.float32), 0.0)

    h1w = h1w_ref[...]
    m1 = h1w[:, 0:D]
    for k in range(1, N0):
        m1 = m1 + h1w[:, k * D:(k + 1) * D]
    m1 = m1 * (1.0 / N0)

    x0 = jnp.concatenate([h0_ref[...], m1], axis=1)
    out0 = jnp.maximum(
        jnp.dot(x0, w0_ref[...], preferred_element_type=jnp.float32), 0.0)

    mo1 = jnp.dot(m_ref[...], out1, preferred_element_type=jnp.float32)

    y = jnp.concatenate([out0, mo1], axis=1)
    out_ref[...] = jnp.dot(y, w1_ref[...], preferred_element_type=jnp.float32)


@jax.jit
def kernel(h0, h1, h2, W_self_0, W_neigh_0, W_self_1, W_neigh_1):
    h1w = h1.reshape(B, N0 * D)
    h2w = h2.reshape(B * N0, N1 * D)
    w0 = jnp.concatenate([W_self_0, W_neigh_0], axis=0)
    w1 = jnp.concatenate([W_self_1, W_neigh_1], axis=0)
    m = jnp.repeat(jnp.eye(R, dtype=jnp.float32), N0, axis=1) * (1.0 / N0)

    grid = (B // R,)
    return pl.pallas_call(
        _fused_sage_kernel,
        grid=grid,
        in_specs=[
            pl.BlockSpec((R, D), lambda i: (i, 0)),
            pl.BlockSpec((R * N0, D), lambda i: (i, 0)),
            pl.BlockSpec((R, N0 * D), lambda i: (i, 0)),
            pl.BlockSpec((R * N0, N1 * D), lambda i: (i, 0)),
            pl.BlockSpec((2 * D, D), lambda i: (0, 0)),
            pl.BlockSpec((2 * D, D), lambda i: (0, 0)),
            pl.BlockSpec((R, R * N0), lambda i: (0, 0)),
        ],
        out_specs=pl.BlockSpec((R, D), lambda i: (i, 0)),
        out_shape=jax.ShapeDtypeStruct((B, D), jnp.float32),
        compiler_params=pltpu.CompilerParams(
            dimension_semantics=("arbitrary",)),
    )(h0, h1, h1w, h2w, w0, w1, m)


# speedup vs baseline: 1.7500x; 1.1045x over previous
import functools

import jax
import jax.numpy as jnp
from jax.experimental import pallas as pl
from jax.experimental.pallas import tpu as pltpu

B = 2048
N0 = 10
N1 = 10
D = 256
R = 64


def _fused_sage_kernel(h0_ref, h1f_ref, h2w_ref, w0_ref, w1_ref,
                       m_ref, out_ref):
    h2w = h2w_ref[...]
    m2 = h2w[:, 0:D]
    for k in range(1, N1):
        m2 = m2 + h2w[:, k * D:(k + 1) * D]
    m2 = m2 * (1.0 / N1)

    x1 = jnp.concatenate([h1f_ref[...], m2], axis=1)
    out1 = jnp.maximum(
        jnp.dot(x1, w0_ref[...], preferred_element_type=jnp.float32), 0.0)

    m1 = jnp.dot(m_ref[...], h1f_ref[...], preferred_element_type=jnp.float32)

    x0 = jnp.concatenate([h0_ref[...], m1], axis=1)
    out0 = jnp.maximum(
        jnp.dot(x0, w0_ref[...], preferred_element_type=jnp.float32), 0.0)

    mo1 = jnp.dot(m_ref[...], out1, preferred_element_type=jnp.float32)

    y = jnp.concatenate([out0, mo1], axis=1)
    out_ref[...] = jnp.dot(y, w1_ref[...], preferred_element_type=jnp.float32)


@jax.jit
def kernel(h0, h1, h2, W_self_0, W_neigh_0, W_self_1, W_neigh_1):
    h2w = h2.reshape(B * N0, N1 * D)
    w0 = jnp.concatenate([W_self_0, W_neigh_0], axis=0)
    w1 = jnp.concatenate([W_self_1, W_neigh_1], axis=0)
    m = jnp.repeat(jnp.eye(R, dtype=jnp.float32), N0, axis=1) * (1.0 / N0)

    grid = (B // R,)
    return pl.pallas_call(
        _fused_sage_kernel,
        grid=grid,
        in_specs=[
            pl.BlockSpec((R, D), lambda i: (i, 0)),
            pl.BlockSpec((R * N0, D), lambda i: (i, 0)),
            pl.BlockSpec((R * N0, N1 * D), lambda i: (i, 0)),
            pl.BlockSpec((2 * D, D), lambda i: (0, 0)),
            pl.BlockSpec((2 * D, D), lambda i: (0, 0)),
            pl.BlockSpec((R, R * N0), lambda i: (0, 0)),
        ],
        out_specs=pl.BlockSpec((R, D), lambda i: (i, 0)),
        out_shape=jax.ShapeDtypeStruct((B, D), jnp.float32),
        compiler_params=pltpu.CompilerParams(
            dimension_semantics=("arbitrary",)),
    )(h0, h1, h2w, w0, w1, m)
